# Initial kernel scaffold; baseline (speedup 1.0000x reference)
import jax
import jax.numpy as jnp
from jax.experimental import pallas as pl


def kernel(x, edge_index, W1, b1, W2, b2):
    raise NotImplementedError("write your pallas kernel here")


# speedup vs baseline: 16.0548x; 16.0548x over previous
import functools

import jax
import jax.numpy as jnp
from jax import lax
from jax.experimental import pallas as pl
from jax.experimental.pallas import tpu as pltpu
from jax.experimental.pallas import tpu_sc as plsc

N = 10000
E = 320000
IN_D = 128
HID = 16
OUT_D = 4

NC, NS = 2, 16
NW = NC * NS
E_PER_W = E // NW
CHUNK = 80
NCHUNK = E_PER_W // CHUNK
NP = 10240
RPS = NP // NS

_MESH = plsc.VectorSubcoreMesh(core_axis_name="c", subcore_axis_name="s")
_SC_PARAMS = pltpu.CompilerParams(use_tc_tiling_on_sc=False)


def _zero_vmem(buf, nrows):
    def body(i, carry):
        buf[i] = jnp.zeros((16,), jnp.float32)
        return carry
    lax.fori_loop(0, nrows, body, 0)


@functools.partial(
    pl.kernel,
    out_type=jax.ShapeDtypeStruct((NC, NP, HID), jnp.float32),
    mesh=_MESH,
    scratch_types=[
        pltpu.VMEM((CHUNK,), jnp.int32),
        pltpu.VMEM((CHUNK,), jnp.int32),
        pltpu.VMEM((CHUNK, HID), jnp.float32),
        pltpu.VMEM((RPS, HID), jnp.float32),
        pltpu.VMEM_SHARED((NP, HID), jnp.float32),
        pltpu.SemaphoreType.DMA,
    ],
    compiler_params=_SC_PARAMS,
)
def _msg_pass(table_hbm, src_hbm, dst_hbm, out_hbm,
              sidx, didx, rows, stg, acc_sh, sem):
    c = lax.axis_index("c")
    s = lax.axis_index("s")
    wid = c * NS + s
    _zero_vmem(stg, RPS)
    pltpu.sync_copy(stg, acc_sh.at[pl.ds(s * RPS, RPS)])
    plsc.subcore_barrier()

    def body(j, carry):
        off = wid * E_PER_W + j * CHUNK
        pltpu.sync_copy(src_hbm.at[pl.ds(off, CHUNK)], sidx)
        pltpu.sync_copy(dst_hbm.at[pl.ds(off, CHUNK)], didx)
        pltpu.async_copy(table_hbm.at[sidx], rows, sem).wait()
        pltpu.sync_copy(rows, acc_sh.at[didx], add=True)
        return carry
    lax.fori_loop(0, NCHUNK, body, 0)

    plsc.subcore_barrier()
    pltpu.sync_copy(acc_sh.at[pl.ds(s * RPS, RPS)], stg)
    pltpu.sync_copy(stg, out_hbm.at[c, pl.ds(s * RPS, RPS)])


@functools.partial(
    pl.kernel,
    out_type=jax.ShapeDtypeStruct((NC, NP, HID), jnp.float32),
    mesh=_MESH,
    scratch_types=[
        pltpu.VMEM((CHUNK,), jnp.int32),
        pltpu.VMEM((CHUNK, HID), jnp.float32),
        pltpu.VMEM((RPS, HID), jnp.float32),
        pltpu.VMEM_SHARED((NP, HID), jnp.float32),
    ],
    compiler_params=_SC_PARAMS,
)
def _deg_pass(dst_hbm, out_hbm, didx, ones, stg, acc_sh):
    c = lax.axis_index("c")
    s = lax.axis_index("s")
    wid = c * NS + s
    _zero_vmem(stg, RPS)
    pltpu.sync_copy(stg, acc_sh.at[pl.ds(s * RPS, RPS)])

    def fill(i, carry):
        ones[i] = jnp.ones((16,), jnp.float32)
        return carry
    lax.fori_loop(0, CHUNK, fill, 0)
    plsc.subcore_barrier()

    def body(j, carry):
        off = wid * E_PER_W + j * CHUNK
        pltpu.sync_copy(dst_hbm.at[pl.ds(off, CHUNK)], didx)
        pltpu.sync_copy(ones, acc_sh.at[didx], add=True)
        return carry
    lax.fori_loop(0, NCHUNK, body, 0)

    plsc.subcore_barrier()
    pltpu.sync_copy(acc_sh.at[pl.ds(s * RPS, RPS)], stg)
    pltpu.sync_copy(stg, out_hbm.at[c, pl.ds(s * RPS, RPS)])


_RB = 1000
_GRID = N // _RB


def _dinv_of(dacc0, dacc1):
    deg = dacc0[:, 0:1] + dacc1[:, 0:1] + 1.0
    return lax.rsqrt(deg)


def _tc1_body(x_ref, w1_ref, dacc_ref, g1_ref):
    dinv = _dinv_of(dacc_ref[0], dacc_ref[1])
    h = jnp.dot(x_ref[...], w1_ref[...], preferred_element_type=jnp.float32)
    g1_ref[...] = h * dinv


def _tc1(x, W1, dacc):
    return pl.pallas_call(
        _tc1_body,
        grid=(_GRID,),
        in_specs=[
            pl.BlockSpec((_RB, IN_D), lambda i: (i, 0)),
            pl.BlockSpec((IN_D, HID), lambda i: (0, 0)),
            pl.BlockSpec((NC, _RB, HID), lambda i: (0, i, 0)),
        ],
        out_specs=pl.BlockSpec((_RB, HID), lambda i: (i, 0)),
        out_shape=jax.ShapeDtypeStruct((N, HID), jnp.float32),
    )(x, W1, dacc)


def _tc2_body(acc_ref, g1_ref, dacc_ref, b1_ref, g2_ref):
    dinv = _dinv_of(dacc_ref[0], dacc_ref[1])
    tot = acc_ref[0] + acc_ref[1] + g1_ref[...]
    z = jnp.maximum(dinv * tot + b1_ref[...], 0.0)
    g2_ref[...] = dinv * z


def _tc2(acc1, g1, dacc, b1):
    return pl.pallas_call(
        _tc2_body,
        grid=(_GRID,),
        in_specs=[
            pl.BlockSpec((NC, _RB, HID), lambda i: (0, i, 0)),
            pl.BlockSpec((_RB, HID), lambda i: (i, 0)),
            pl.BlockSpec((NC, _RB, HID), lambda i: (0, i, 0)),
            pl.BlockSpec((1, HID), lambda i: (0, 0)),
        ],
        out_specs=pl.BlockSpec((_RB, HID), lambda i: (i, 0)),
        out_shape=jax.ShapeDtypeStruct((N, HID), jnp.float32),
    )(acc1, g1, dacc, b1.reshape(1, HID))


def _tc3_body(acc_ref, g2_ref, dacc_ref, w2_ref, b2_ref, out_ref):
    dinv = _dinv_of(dacc_ref[0], dacc_ref[1])
    t = dinv * (acc_ref[0] + acc_ref[1] + g2_ref[...])
    h = jnp.dot(t, w2_ref[...], preferred_element_type=jnp.float32)
    h = h + b2_ref[...]
    m = jnp.max(h, axis=1, keepdims=True)
    e = jnp.exp(h - m)
    lse = jnp.log(jnp.sum(e, axis=1, keepdims=True))
    out_ref[...] = h - m - lse


def _tc3(acc2, g2, dacc, W2, b2):
    return pl.pallas_call(
        _tc3_body,
        grid=(_GRID,),
        in_specs=[
            pl.BlockSpec((NC, _RB, HID), lambda i: (0, i, 0)),
            pl.BlockSpec((_RB, HID), lambda i: (i, 0)),
            pl.BlockSpec((NC, _RB, HID), lambda i: (0, i, 0)),
            pl.BlockSpec((HID, OUT_D), lambda i: (0, 0)),
            pl.BlockSpec((1, OUT_D), lambda i: (0, 0)),
        ],
        out_specs=pl.BlockSpec((_RB, OUT_D), lambda i: (i, 0)),
        out_shape=jax.ShapeDtypeStruct((N, OUT_D), jnp.float32),
    )(acc2, g2, dacc, W2, b2.reshape(1, OUT_D))


def kernel(x, edge_index, W1, b1, W2, b2):
    src = edge_index[0].astype(jnp.int32)
    dst = edge_index[1].astype(jnp.int32)
    dacc = _deg_pass(dst)
    g1 = _tc1(x, W1, dacc)
    acc1 = _msg_pass(g1, src, dst)
    g2 = _tc2(acc1, g1, dacc, b1)
    acc2 = _msg_pass(g2, src, dst)
    return _tc3(acc2, g2, dacc, W2, b2)


# speedup vs baseline: 46.7912x; 2.9145x over previous
import functools

import jax
import jax.numpy as jnp
from jax import lax
from jax.experimental import pallas as pl
from jax.experimental.pallas import tpu as pltpu
from jax.experimental.pallas import tpu_sc as plsc

N = 10000
E = 320000
IN_D = 128
HID = 16
OUT_D = 4

NC, NS = 2, 16
NW = NC * NS
E_PER_W = E // NW
CHUNK = 80
NCHUNK = E_PER_W // CHUNK
NP = 10240
RPS = NP // NS

_MESH = plsc.VectorSubcoreMesh(core_axis_name="c", subcore_axis_name="s")
_SC_PARAMS = pltpu.CompilerParams(use_tc_tiling_on_sc=False)


def _zero_vmem(buf, nrows):
    def body(i, carry):
        buf[i] = jnp.zeros((16,), jnp.float32)
        return carry
    lax.fori_loop(0, nrows, body, 0)


NBUF = 5
NOUTER = NCHUNK // NBUF


@functools.partial(
    pl.kernel,
    out_type=jax.ShapeDtypeStruct((NC, NP, HID), jnp.float32),
    mesh=_MESH,
    scratch_types=[
        pltpu.VMEM((E_PER_W,), jnp.int32),
        [pltpu.VMEM((CHUNK,), jnp.int32) for _ in range(NBUF)],
        [pltpu.VMEM((CHUNK, HID), jnp.float32) for _ in range(NBUF)],
        pltpu.VMEM((RPS, HID), jnp.float32),
        pltpu.VMEM_SHARED((NP, HID), jnp.float32),
        pltpu.SemaphoreType.DMA((NBUF,)),
        pltpu.SemaphoreType.DMA((NBUF,)),
        pltpu.SemaphoreType.DMA((NBUF,)),
    ],
    compiler_params=_SC_PARAMS,
)
def _msg_pass(table_hbm, src_hbm, dst_hbm, out_hbm,
              sidx_all, didx, rows, stg, acc_sh, sem_i, sem_g, sem_s):
    c = lax.axis_index("c")
    s = lax.axis_index("s")
    wid = c * NS + s
    base = wid * E_PER_W
    _zero_vmem(stg, RPS)
    pltpu.sync_copy(stg, acc_sh.at[pl.ds(s * RPS, RPS)])
    pltpu.sync_copy(src_hbm.at[pl.ds(base, E_PER_W)], sidx_all)
    plsc.subcore_barrier()

    def gather_of(j, b):
        return pltpu.make_async_copy(
            table_hbm.at[sidx_all.at[pl.ds(j * CHUNK, CHUNK)]],
            rows[b], sem_g.at[b])

    def idx_of(j, b):
        return pltpu.make_async_copy(
            dst_hbm.at[pl.ds(base + j * CHUNK, CHUNK)], didx[b], sem_i.at[b])

    def scat_of(b):
        return pltpu.make_async_copy(rows[b], acc_sh.at[didx[b]], sem_s.at[b])

    for b in range(NBUF):
        idx_of(b, b).start()
        gather_of(b, b).start()

    def fire(j0, carry):
        for b in range(NBUF):
            idx_of(j0 + b, b).wait()
            gather_of(j0 + b, b).wait()
            pltpu.async_copy(rows[b], acc_sh.at[didx[b]], sem_s.at[b],
                             add=True)
        return carry

    def outer(g, carry):
        j0 = g * NBUF
        fire(j0, 0)
        for b in range(NBUF):
            scat_of(b).wait()
            idx_of(j0 + NBUF + b, b).start()
            gather_of(j0 + NBUF + b, b).start()
        return carry
    lax.fori_loop(0, NOUTER - 1, outer, 0)

    fire((NOUTER - 1) * NBUF, 0)
    for b in range(NBUF):
        scat_of(b).wait()

    plsc.subcore_barrier()
    pltpu.sync_copy(acc_sh.at[pl.ds(s * RPS, RPS)], stg)
    pltpu.sync_copy(stg, out_hbm.at[c, pl.ds(s * RPS, RPS)])


@functools.partial(
    pl.kernel,
    out_type=jax.ShapeDtypeStruct((NC, NP, HID), jnp.float32),
    mesh=_MESH,
    scratch_types=[
        [pltpu.VMEM((CHUNK,), jnp.int32) for _ in range(NBUF)],
        pltpu.VMEM((CHUNK, HID), jnp.float32),
        pltpu.VMEM((RPS, HID), jnp.float32),
        pltpu.VMEM_SHARED((NP, HID), jnp.float32),
        pltpu.SemaphoreType.DMA((NBUF,)),
        pltpu.SemaphoreType.DMA((NBUF,)),
    ],
    compiler_params=_SC_PARAMS,
)
def _deg_pass(dst_hbm, out_hbm, didx, ones, stg, acc_sh, sem_i, sem_s):
    c = lax.axis_index("c")
    s = lax.axis_index("s")
    wid = c * NS + s
    base = wid * E_PER_W
    _zero_vmem(stg, RPS)
    pltpu.sync_copy(stg, acc_sh.at[pl.ds(s * RPS, RPS)])

    def fill(i, carry):
        ones[i] = jnp.ones((16,), jnp.float32)
        return carry
    lax.fori_loop(0, CHUNK, fill, 0)
    plsc.subcore_barrier()

    def idx_of(j, b):
        return pltpu.make_async_copy(
            dst_hbm.at[pl.ds(base + j * CHUNK, CHUNK)], didx[b], sem_i.at[b])

    def scat_of(b):
        return pltpu.make_async_copy(ones, acc_sh.at[didx[b]], sem_s.at[b])

    for b in range(NBUF):
        idx_of(b, b).start()

    def fire(j0, carry):
        for b in range(NBUF):
            idx_of(j0 + b, b).wait()
            pltpu.async_copy(ones, acc_sh.at[didx[b]], sem_s.at[b], add=True)
        return carry

    def outer(g, carry):
        j0 = g * NBUF
        fire(j0, 0)
        for b in range(NBUF):
            scat_of(b).wait()
            idx_of(j0 + NBUF + b, b).start()
        return carry
    lax.fori_loop(0, NOUTER - 1, outer, 0)

    fire((NOUTER - 1) * NBUF, 0)
    for b in range(NBUF):
        scat_of(b).wait()

    plsc.subcore_barrier()
    pltpu.sync_copy(acc_sh.at[pl.ds(s * RPS, RPS)], stg)
    pltpu.sync_copy(stg, out_hbm.at[c, pl.ds(s * RPS, RPS)])


_RB = 1000
_GRID = N // _RB


def _dinv_of(dacc0, dacc1):
    deg = dacc0[:, 0:1] + dacc1[:, 0:1] + 1.0
    return lax.rsqrt(deg)


def _tcmm_body(x_ref, w1_ref, h_ref):
    h_ref[...] = jnp.dot(x_ref[...], w1_ref[...],
                         preferred_element_type=jnp.float32)


def _tcmm(x, W1):
    return pl.pallas_call(
        _tcmm_body,
        grid=(_GRID,),
        in_specs=[
            pl.BlockSpec((_RB, IN_D), lambda i: (i, 0)),
            pl.BlockSpec((IN_D, HID), lambda i: (0, 0)),
        ],
        out_specs=pl.BlockSpec((_RB, HID), lambda i: (i, 0)),
        out_shape=jax.ShapeDtypeStruct((N, HID), jnp.float32),
    )(x, W1)


def _tc1_body(h_ref, dacc_ref, g1_ref):
    dinv = _dinv_of(dacc_ref[0], dacc_ref[1])
    g1_ref[...] = h_ref[...] * dinv


def _tc1(h, dacc):
    return pl.pallas_call(
        _tc1_body,
        grid=(_GRID,),
        in_specs=[
            pl.BlockSpec((_RB, HID), lambda i: (i, 0)),
            pl.BlockSpec((NC, _RB, HID), lambda i: (0, i, 0)),
        ],
        out_specs=pl.BlockSpec((_RB, HID), lambda i: (i, 0)),
        out_shape=jax.ShapeDtypeStruct((N, HID), jnp.float32),
    )(h, dacc)


def _tc2_body(acc_ref, g1_ref, dacc_ref, b1_ref, g2_ref):
    dinv = _dinv_of(dacc_ref[0], dacc_ref[1])
    tot = acc_ref[0] + acc_ref[1] + g1_ref[...]
    z = jnp.maximum(dinv * tot + b1_ref[...], 0.0)
    g2_ref[...] = dinv * z


def _tc2(acc1, g1, dacc, b1):
    return pl.pallas_call(
        _tc2_body,
        grid=(_GRID,),
        in_specs=[
            pl.BlockSpec((NC, _RB, HID), lambda i: (0, i, 0)),
            pl.BlockSpec((_RB, HID), lambda i: (i, 0)),
            pl.BlockSpec((NC, _RB, HID), lambda i: (0, i, 0)),
            pl.BlockSpec((1, HID), lambda i: (0, 0)),
        ],
        out_specs=pl.BlockSpec((_RB, HID), lambda i: (i, 0)),
        out_shape=jax.ShapeDtypeStruct((N, HID), jnp.float32),
    )(acc1, g1, dacc, b1.reshape(1, HID))


def _tc3_body(acc_ref, g2_ref, dacc_ref, w2_ref, b2_ref, out_ref):
    dinv = _dinv_of(dacc_ref[0], dacc_ref[1])
    t = dinv * (acc_ref[0] + acc_ref[1] + g2_ref[...])
    h = jnp.dot(t, w2_ref[...], preferred_element_type=jnp.float32)
    h = h + b2_ref[...]
    m = jnp.max(h, axis=1, keepdims=True)
    e = jnp.exp(h - m)
    lse = jnp.log(jnp.sum(e, axis=1, keepdims=True))
    out_ref[...] = h - m - lse


def _tc3(acc2, g2, dacc, W2, b2):
    return pl.pallas_call(
        _tc3_body,
        grid=(_GRID,),
        in_specs=[
            pl.BlockSpec((NC, _RB, HID), lambda i: (0, i, 0)),
            pl.BlockSpec((_RB, HID), lambda i: (i, 0)),
            pl.BlockSpec((NC, _RB, HID), lambda i: (0, i, 0)),
            pl.BlockSpec((HID, OUT_D), lambda i: (0, 0)),
            pl.BlockSpec((1, OUT_D), lambda i: (0, 0)),
        ],
        out_specs=pl.BlockSpec((_RB, OUT_D), lambda i: (i, 0)),
        out_shape=jax.ShapeDtypeStruct((N, OUT_D), jnp.float32),
    )(acc2, g2, dacc, W2, b2.reshape(1, OUT_D))


def kernel(x, edge_index, W1, b1, W2, b2):
    src = edge_index[0].astype(jnp.int32)
    dst = edge_index[1].astype(jnp.int32)
    dacc = _deg_pass(dst)
    h = _tcmm(x, W1)
    g1 = _tc1(h, dacc)
    acc1 = _msg_pass(g1, src, dst)
    g2 = _tc2(acc1, g1, dacc, b1)
    acc2 = _msg_pass(g2, src, dst)
    return _tc3(acc2, g2, dacc, W2, b2)


# speedup vs baseline: 66.5041x; 1.4213x over previous
import functools

import jax
import jax.numpy as jnp
from jax import lax
from jax.experimental import pallas as pl
from jax.experimental.pallas import tpu as pltpu
from jax.experimental.pallas import tpu_sc as plsc

N = 10000
E = 320000
IN_D = 128
HID = 16
OUT_D = 4
NW8 = N // 8

NC, NS = 2, 16
NW = NC * NS
E_PER_W = E // NW
CHUNK = 80
NCHUNK = E_PER_W // CHUNK
RPS = N // NS
NBUF = 5
NOUTER = NCHUNK // NBUF

_MESH = plsc.VectorSubcoreMesh(core_axis_name="c", subcore_axis_name="s",
                               num_cores=NC, num_subcores=NS)
_SC_PARAMS = pltpu.CompilerParams(use_tc_tiling_on_sc=False)


def _zero_vmem(buf, nrows):
    def body(i, carry):
        buf[i] = jnp.zeros((16,), jnp.float32)
        return carry
    lax.fori_loop(0, nrows, body, 0)


@functools.partial(
    pl.kernel,
    out_type=jax.ShapeDtypeStruct((NC, N, HID), jnp.float32),
    mesh=_MESH,
    scratch_types=[
        pltpu.VMEM((E_PER_W,), jnp.int32),
        [pltpu.VMEM((CHUNK,), jnp.int32) for _ in range(NBUF)],
        [pltpu.VMEM((CHUNK, HID), jnp.float32) for _ in range(NBUF)],
        pltpu.VMEM((RPS, HID), jnp.float32),
        pltpu.VMEM_SHARED((N, HID), jnp.float32),
        pltpu.SemaphoreType.DMA((NBUF,)),
        pltpu.SemaphoreType.DMA((NBUF,)),
        pltpu.SemaphoreType.DMA((NBUF,)),
    ],
    compiler_params=_SC_PARAMS,
)
def _msg_pass(table_hbm, ei_hbm, out_hbm,
              sidx_all, didx, rows, stg, acc_sh, sem_i, sem_g, sem_s):
    c = lax.axis_index("c")
    s = lax.axis_index("s")
    wid = c * NS + s
    base = wid * E_PER_W
    _zero_vmem(stg, RPS)
    pltpu.sync_copy(stg, acc_sh.at[pl.ds(s * RPS, RPS)])
    pltpu.sync_copy(ei_hbm.at[0, pl.ds(base, E_PER_W)], sidx_all)
    plsc.subcore_barrier()

    def gather_of(j, b):
        return pltpu.make_async_copy(
            table_hbm.at[sidx_all.at[pl.ds(j * CHUNK, CHUNK)]],
            rows[b], sem_g.at[b])

    def idx_of(j, b):
        return pltpu.make_async_copy(
            ei_hbm.at[1, pl.ds(base + j * CHUNK, CHUNK)], didx[b],
            sem_i.at[b])

    def scat_of(b):
        return pltpu.make_async_copy(rows[b], acc_sh.at[didx[b]], sem_s.at[b])

    for b in range(NBUF):
        idx_of(b, b).start()
        gather_of(b, b).start()

    def fire(j0, carry):
        for b in range(NBUF):
            idx_of(j0 + b, b).wait()
            gather_of(j0 + b, b).wait()
            pltpu.async_copy(rows[b], acc_sh.at[didx[b]], sem_s.at[b],
                             add=True)
        return carry

    def outer(g, carry):
        j0 = g * NBUF
        fire(j0, 0)
        for b in range(NBUF):
            scat_of(b).wait()
            idx_of(j0 + NBUF + b, b).start()
            gather_of(j0 + NBUF + b, b).start()
        return carry
    lax.fori_loop(0, NOUTER - 1, outer, 0)

    fire((NOUTER - 1) * NBUF, 0)
    for b in range(NBUF):
        scat_of(b).wait()

    plsc.subcore_barrier()
    pltpu.sync_copy(acc_sh.at[pl.ds(s * RPS, RPS)], stg)
    pltpu.sync_copy(stg, out_hbm.at[c, pl.ds(s * RPS, RPS)])


@functools.partial(
    pl.kernel,
    out_type=jax.ShapeDtypeStruct((NC, N, HID), jnp.float32),
    mesh=_MESH,
    scratch_types=[
        [pltpu.VMEM((CHUNK,), jnp.int32) for _ in range(NBUF)],
        pltpu.VMEM((CHUNK, HID), jnp.float32),
        pltpu.VMEM((RPS, HID), jnp.float32),
        pltpu.VMEM_SHARED((N, HID), jnp.float32),
        pltpu.SemaphoreType.DMA((NBUF,)),
        pltpu.SemaphoreType.DMA((NBUF,)),
    ],
    compiler_params=_SC_PARAMS,
)
def _deg_pass(ei_hbm, out_hbm, didx, ones, stg, acc_sh, sem_i, sem_s):
    c = lax.axis_index("c")
    s = lax.axis_index("s")
    wid = c * NS + s
    base = wid * E_PER_W
    _zero_vmem(stg, RPS)
    pltpu.sync_copy(stg, acc_sh.at[pl.ds(s * RPS, RPS)])

    def fill(i, carry):
        ones[i] = jnp.ones((16,), jnp.float32)
        return carry
    lax.fori_loop(0, CHUNK, fill, 0)
    plsc.subcore_barrier()

    def idx_of(j, b):
        return pltpu.make_async_copy(
            ei_hbm.at[1, pl.ds(base + j * CHUNK, CHUNK)], didx[b],
            sem_i.at[b])

    def scat_of(b):
        return pltpu.make_async_copy(ones, acc_sh.at[didx[b]], sem_s.at[b])

    for b in range(NBUF):
        idx_of(b, b).start()

    def fire(j0, carry):
        for b in range(NBUF):
            idx_of(j0 + b, b).wait()
            pltpu.async_copy(ones, acc_sh.at[didx[b]], sem_s.at[b], add=True)
        return carry

    def outer(g, carry):
        j0 = g * NBUF
        fire(j0, 0)
        for b in range(NBUF):
            scat_of(b).wait()
            idx_of(j0 + NBUF + b, b).start()
        return carry
    lax.fori_loop(0, NOUTER - 1, outer, 0)

    fire((NOUTER - 1) * NBUF, 0)
    for b in range(NBUF):
        scat_of(b).wait()

    plsc.subcore_barrier()
    pltpu.sync_copy(acc_sh.at[pl.ds(s * RPS, RPS)], stg)
    pltpu.sync_copy(stg, out_hbm.at[c, pl.ds(s * RPS, RPS)])


def _dinv_wide(dacc_ref):
    return lax.rsqrt(dacc_ref[0] + dacc_ref[1] + 1.0)


def _tca_body(x_ref, w1_ref, dacc_ref, g1_ref):
    w1 = w1_ref[...]
    parts = [
        jnp.dot(x_ref[:, k, :], w1, preferred_element_type=jnp.float32)
        for k in range(8)
    ]
    g1_ref[...] = jnp.concatenate(parts, axis=1) * _dinv_wide(dacc_ref)


def _tca(x, W1, dacc_w):
    return pl.pallas_call(
        _tca_body,
        out_shape=jax.ShapeDtypeStruct((NW8, 128), jnp.float32),
    )(x.reshape(NW8, 8, IN_D), W1, dacc_w)


def _tcb_body(acc_ref, g1_ref, dacc_ref, b1_ref, g2_ref):
    dinv = _dinv_wide(dacc_ref)
    z = jnp.maximum(dinv * (acc_ref[0] + acc_ref[1] + g1_ref[...])
                    + b1_ref[...], 0.0)
    g2_ref[...] = dinv * z


def _tcb(acc1_w, g1_w, dacc_w, b1):
    return pl.pallas_call(
        _tcb_body,
        out_shape=jax.ShapeDtypeStruct((NW8, 128), jnp.float32),
    )(acc1_w, g1_w, dacc_w, jnp.tile(b1, 8).reshape(1, 128))


def _tcc_body(acc_ref, g2_ref, dacc_ref, w2b_ref, b2w_ref, savg_ref,
              ssum_ref, out_ref):
    dinv = _dinv_wide(dacc_ref)
    t_w = dinv * (acc_ref[0] + acc_ref[1] + g2_ref[...])
    h = jnp.dot(t_w, w2b_ref[...], preferred_element_type=jnp.float32)
    h = h + b2w_ref[...]
    m = jnp.dot(h, savg_ref[...], preferred_element_type=jnp.float32)
    e = jnp.exp(h - m)
    lse = jnp.log(jnp.dot(e, ssum_ref[...],
                          preferred_element_type=jnp.float32))
    out_ref[...] = h - m - lse


def _tcc(acc2_w, g2_w, dacc_w, W2, b2):
    eye8 = jnp.eye(8, dtype=jnp.float32)
    w2big = jnp.kron(eye8, W2)
    savg = jnp.kron(eye8, jnp.full((OUT_D, OUT_D), 0.25, jnp.float32))
    ssum = jnp.kron(eye8, jnp.ones((OUT_D, OUT_D), jnp.float32))
    b2w = jnp.tile(b2, 8).reshape(1, 8 * OUT_D)
    out_w = pl.pallas_call(
        _tcc_body,
        out_shape=jax.ShapeDtypeStruct((NW8, 8 * OUT_D), jnp.float32),
    )(acc2_w, g2_w, dacc_w, w2big, b2w, savg, ssum)
    return out_w.reshape(N, OUT_D)


def kernel(x, edge_index, W1, b1, W2, b2):
    ei = edge_index.astype(jnp.int32)
    dacc_w = _deg_pass(ei).reshape(NC, NW8, 128)
    g1_w = _tca(x, W1, dacc_w)
    acc1_w = _msg_pass(g1_w.reshape(N, HID), ei).reshape(NC, NW8, 128)
    g2_w = _tcb(acc1_w, g1_w, dacc_w, b1)
    acc2_w = _msg_pass(g2_w.reshape(N, HID), ei).reshape(NC, NW8, 128)
    return _tcc(acc2_w, g2_w, dacc_w, W2, b2)


# speedup vs baseline: 81.0538x; 1.2188x over previous
import functools

import jax
import jax.numpy as jnp
from jax import lax
from jax.experimental import pallas as pl
from jax.experimental.pallas import tpu as pltpu
from jax.experimental.pallas import tpu_sc as plsc

N = 10000
E = 320000
IN_D = 128
HID = 16
OUT_D = 4
NW8 = N // 8

NC, NS = 2, 16
NW = NC * NS
E_PER_W = E // NW
CHUNK = 80
NCHUNK = E_PER_W // CHUNK
RPS = N // NS
NBUF = 8
NFULL = NCHUNK // NBUF
NREM = NCHUNK % NBUF

_MESH = plsc.VectorSubcoreMesh(core_axis_name="c", subcore_axis_name="s",
                               num_cores=NC, num_subcores=NS)
_SC_PARAMS = pltpu.CompilerParams(use_tc_tiling_on_sc=False)


def _zero_vmem(buf, nrows):
    def body(i, carry):
        buf[i] = jnp.zeros((16,), jnp.float32)
        return carry
    lax.fori_loop(0, nrows, body, 0)


@functools.partial(
    pl.kernel,
    out_type=jax.ShapeDtypeStruct((NC, N, HID), jnp.float32),
    mesh=_MESH,
    scratch_types=[
        pltpu.VMEM((E_PER_W,), jnp.int32),
        [pltpu.VMEM((CHUNK,), jnp.int32) for _ in range(NBUF)],
        [pltpu.VMEM((CHUNK, HID), jnp.float32) for _ in range(NBUF)],
        pltpu.VMEM((RPS, HID), jnp.float32),
        pltpu.VMEM_SHARED((N, HID), jnp.float32),
        pltpu.SemaphoreType.DMA((NBUF,)),
        pltpu.SemaphoreType.DMA((NBUF,)),
    ],
    compiler_params=_SC_PARAMS,
)
def _msg_pass(table_hbm, ei_hbm, out_hbm,
              sidx_all, didx, rows, stg, acc_sh, sem_ig, sem_s):
    c = lax.axis_index("c")
    s = lax.axis_index("s")
    wid = c * NS + s
    base = wid * E_PER_W
    _zero_vmem(stg, RPS)
    pltpu.sync_copy(stg, acc_sh.at[pl.ds(s * RPS, RPS)])
    pltpu.sync_copy(ei_hbm.at[0, pl.ds(base, E_PER_W)], sidx_all)
    plsc.subcore_barrier()

    def gather_of(j, b):
        return pltpu.make_async_copy(
            table_hbm.at[sidx_all.at[pl.ds(j * CHUNK, CHUNK)]],
            rows[b], sem_ig.at[b])

    def idx_of(j, b):
        return pltpu.make_async_copy(
            ei_hbm.at[1, pl.ds(base + j * CHUNK, CHUNK)], didx[b],
            sem_ig.at[b])

    def scat_of(b):
        return pltpu.make_async_copy(rows[b], acc_sh.at[didx[b]], sem_s.at[b])

    def fetch(j, b):
        idx_of(j, b).start()
        gather_of(j, b).start()

    def fire(j, b):
        idx_of(j, b).wait()
        gather_of(j, b).wait()
        pltpu.async_copy(rows[b], acc_sh.at[didx[b]], sem_s.at[b], add=True)

    for b in range(NBUF):
        fetch(b, b)

    def outer(g, carry):
        j0 = g * NBUF
        for b in range(NBUF):
            fire(j0 + b, b)
        for b in range(NBUF):
            scat_of(b).wait()
            fetch(j0 + NBUF + b, b)
        return carry
    lax.fori_loop(0, NFULL - 1, outer, 0)

    j0 = (NFULL - 1) * NBUF
    for b in range(NBUF):
        fire(j0 + b, b)
    for b in range(NREM):
        scat_of(b).wait()
        fetch(NFULL * NBUF + b, b)
    for b in range(NREM):
        fire(NFULL * NBUF + b, b)
    for b in range(NBUF):
        scat_of(b).wait()

    plsc.subcore_barrier()
    pltpu.sync_copy(acc_sh.at[pl.ds(s * RPS, RPS)], stg)
    pltpu.sync_copy(stg, out_hbm.at[c, pl.ds(s * RPS, RPS)])


@functools.partial(
    pl.kernel,
    out_type=jax.ShapeDtypeStruct((NC, N, HID), jnp.float32),
    mesh=_MESH,
    scratch_types=[
        [pltpu.VMEM((CHUNK,), jnp.int32) for _ in range(NBUF)],
        pltpu.VMEM((CHUNK, HID), jnp.float32),
        pltpu.VMEM((RPS, HID), jnp.float32),
        pltpu.VMEM_SHARED((N, HID), jnp.float32),
        pltpu.SemaphoreType.DMA((NBUF,)),
        pltpu.SemaphoreType.DMA((NBUF,)),
    ],
    compiler_params=_SC_PARAMS,
)
def _deg_pass(ei_hbm, out_hbm, didx, ones, stg, acc_sh, sem_i, sem_s):
    c = lax.axis_index("c")
    s = lax.axis_index("s")
    wid = c * NS + s
    base = wid * E_PER_W
    _zero_vmem(stg, RPS)
    pltpu.sync_copy(stg, acc_sh.at[pl.ds(s * RPS, RPS)])

    def fill(i, carry):
        ones[i] = jnp.ones((16,), jnp.float32)
        return carry
    lax.fori_loop(0, CHUNK, fill, 0)
    plsc.subcore_barrier()

    def idx_of(j, b):
        return pltpu.make_async_copy(
            ei_hbm.at[1, pl.ds(base + j * CHUNK, CHUNK)], didx[b],
            sem_i.at[b])

    def scat_of(b):
        return pltpu.make_async_copy(ones, acc_sh.at[didx[b]], sem_s.at[b])

    def fire(j, b):
        idx_of(j, b).wait()
        pltpu.async_copy(ones, acc_sh.at[didx[b]], sem_s.at[b], add=True)

    for b in range(NBUF):
        idx_of(b, b).start()

    def outer(g, carry):
        j0 = g * NBUF
        for b in range(NBUF):
            fire(j0 + b, b)
        for b in range(NBUF):
            scat_of(b).wait()
            idx_of(j0 + NBUF + b, b).start()
        return carry
    lax.fori_loop(0, NFULL - 1, outer, 0)

    j0 = (NFULL - 1) * NBUF
    for b in range(NBUF):
        fire(j0 + b, b)
    for b in range(NREM):
        scat_of(b).wait()
        idx_of(NFULL * NBUF + b, b).start()
    for b in range(NREM):
        fire(NFULL * NBUF + b, b)
    for b in range(NBUF):
        scat_of(b).wait()

    plsc.subcore_barrier()
    pltpu.sync_copy(acc_sh.at[pl.ds(s * RPS, RPS)], stg)
    pltpu.sync_copy(stg, out_hbm.at[c, pl.ds(s * RPS, RPS)])


def _dinv_wide(dacc_ref):
    return lax.rsqrt(dacc_ref[0] + dacc_ref[1] + 1.0)


def _tca_body(x_ref, w1_ref, dacc_ref, g1_ref):
    w1 = w1_ref[...]
    parts = [
        jnp.dot(x_ref[:, k, :], w1, preferred_element_type=jnp.float32)
        for k in range(8)
    ]
    g1_ref[...] = jnp.concatenate(parts, axis=1) * _dinv_wide(dacc_ref)


def _tca(x, W1, dacc_w):
    return pl.pallas_call(
        _tca_body,
        out_shape=jax.ShapeDtypeStruct((NW8, 128), jnp.float32),
    )(x.reshape(NW8, 8, IN_D), W1, dacc_w)


def _tcb_body(acc_ref, g1_ref, dacc_ref, b1_ref, g2_ref):
    dinv = _dinv_wide(dacc_ref)
    z = jnp.maximum(dinv * (acc_ref[0] + acc_ref[1] + g1_ref[...])
                    + b1_ref[...], 0.0)
    g2_ref[...] = dinv * z


def _tcb(acc1_w, g1_w, dacc_w, b1):
    return pl.pallas_call(
        _tcb_body,
        out_shape=jax.ShapeDtypeStruct((NW8, 128), jnp.float32),
    )(acc1_w, g1_w, dacc_w, jnp.tile(b1, 8).reshape(1, 128))


def _tcc_body(acc_ref, g2_ref, dacc_ref, w2b_ref, b2w_ref, savg_ref,
              ssum_ref, out_ref):
    dinv = _dinv_wide(dacc_ref)
    t_w = dinv * (acc_ref[0] + acc_ref[1] + g2_ref[...])
    h = jnp.dot(t_w, w2b_ref[...], preferred_element_type=jnp.float32)
    h = h + b2w_ref[...]
    m = jnp.dot(h, savg_ref[...], preferred_element_type=jnp.float32)
    e = jnp.exp(h - m)
    lse = jnp.log(jnp.dot(e, ssum_ref[...],
                          preferred_element_type=jnp.float32))
    out_ref[...] = h - m - lse


def _tcc(acc2_w, g2_w, dacc_w, W2, b2):
    eye8 = jnp.eye(8, dtype=jnp.float32)
    w2big = jnp.kron(eye8, W2)
    savg = jnp.kron(eye8, jnp.full((OUT_D, OUT_D), 0.25, jnp.float32))
    ssum = jnp.kron(eye8, jnp.ones((OUT_D, OUT_D), jnp.float32))
    b2w = jnp.tile(b2, 8).reshape(1, 8 * OUT_D)
    out_w = pl.pallas_call(
        _tcc_body,
        out_shape=jax.ShapeDtypeStruct((NW8, 8 * OUT_D), jnp.float32),
    )(acc2_w, g2_w, dacc_w, w2big, b2w, savg, ssum)
    return out_w.reshape(N, OUT_D)


def kernel(x, edge_index, W1, b1, W2, b2):
    ei = edge_index.astype(jnp.int32)
    dacc_w = _deg_pass(ei).reshape(NC, NW8, 128)
    g1_w = _tca(x, W1, dacc_w)
    acc1_w = _msg_pass(g1_w.reshape(N, HID), ei).reshape(NC, NW8, 128)
    g2_w = _tcb(acc1_w, g1_w, dacc_w, b1)
    acc2_w = _msg_pass(g2_w.reshape(N, HID), ei).reshape(NC, NW8, 128)
    return _tcc(acc2_w, g2_w, dacc_w, W2, b2)
